# baseline (speedup 1.0000x reference)
import functools

import jax
import jax.numpy as jnp
import numpy as np
from jax import lax
from jax.experimental import pallas as pl
from jax.experimental.pallas import tpu as pltpu
from jax.experimental.pallas import tpu_sc as plsc

VOCAB_N = 100000
D = 1024
POS_N = 2048
BATCH = 4
B = BATCH * POS_N

_info = plsc.get_sparse_core_info()
NC, NS, LANES = _info.num_cores, _info.num_subcores, _info.num_lanes
NW = NC * NS
BPW = B // NW
CH = 16
NCH = BPW // CH
RING = 3
PREF = 2


def _pos_table():
    half = D / 2
    positions = np.arange(POS_N, dtype=np.float32)[:, None]
    depths = np.arange(half, dtype=np.float32)[None, :] / np.float32(half)
    angle_rads = positions * (np.float32(1.0) / (10000.0 ** depths)).astype(np.float32)
    return np.concatenate([np.sin(angle_rads), np.cos(angle_rads)],
                          axis=-1).astype(np.float32)


_POS = _pos_table().reshape(POS_N * 8, 128)


def _sc_body(table_hbm, idx_hbm, pos_hbm, out_hbm, *scr):
    idx_v = scr[0]
    bufs = scr[1:1 + RING]
    pbufs = scr[1 + RING:1 + 2 * RING]
    gsem = scr[1 + 2 * RING:1 + 3 * RING]
    psem = scr[1 + 3 * RING:1 + 4 * RING]
    ssem = scr[1 + 4 * RING:1 + 5 * RING]

    wid = lax.axis_index("s") * NC + lax.axis_index("c")
    base = wid * BPW
    batch = wid // (POS_N // BPW)
    pos_base = (wid % (POS_N // BPW)) * BPW
    pltpu.sync_copy(idx_hbm.at[batch, pl.ds(pos_base, BPW)], idx_v)

    def start_fetch(c):
        s = c % RING
        pltpu.async_copy(
            table_hbm.at[idx_v.at[pl.ds(c * CH, CH)]], bufs[s], gsem[s])
        pltpu.async_copy(pos_hbm.at[pl.ds((pos_base + c * CH) * 8, CH * 8)],
                         pbufs[s], psem[s])

    for c in range(PREF):
        start_fetch(c)

    for c in range(NCH):
        s = c % RING
        pltpu.make_async_copy(table_hbm.at[idx_v.at[pl.ds(c * CH, CH)]],
                              bufs[s], gsem[s]).wait()
        pltpu.make_async_copy(
            pos_hbm.at[pl.ds((pos_base + c * CH) * 8, CH * 8)],
            pbufs[s], psem[s]).wait()

        buf, pbuf = bufs[s], pbufs[s]

        def row(r, carry, buf=buf, pbuf=pbuf):
            r8 = r * 8
            for j in range(D // LANES):
                sl = pl.ds(j * LANES, LANES)
                psl = pl.ds((j % 8) * LANES, LANES)
                buf[r, sl] = buf[r, sl] * 32.0 + pbuf[r8 + j // 8, psl]
            return carry

        lax.fori_loop(0, CH, row, 0)

        pltpu.async_copy(bufs[s], out_hbm.at[pl.ds(base + c * CH, CH)],
                         ssem[s])
        cp = c + PREF
        if cp < NCH:
            if cp >= RING:
                sp = cp % RING
                pltpu.make_async_copy(
                    bufs[sp], out_hbm.at[pl.ds(base + (cp - RING) * CH, CH)],
                    ssem[sp]).wait()
            start_fetch(cp)

    for c in range(NCH - RING, NCH):
        s = c % RING
        pltpu.make_async_copy(bufs[s], out_hbm.at[pl.ds(base + c * CH, CH)],
                              ssem[s]).wait()


@jax.jit
def _sc_embed(table, idx, pos):
    mesh = plsc.VectorSubcoreMesh(core_axis_name="c", subcore_axis_name="s")
    scratch = ([pltpu.VMEM((BPW,), jnp.int32)]
               + [pltpu.VMEM((CH, D), jnp.float32) for _ in range(RING)]
               + [pltpu.VMEM((CH * 8, 128), jnp.float32) for _ in range(RING)]
               + [pltpu.SemaphoreType.DMA for _ in range(3 * RING)])
    f = functools.partial(
        pl.kernel,
        mesh=mesh,
        out_type=jax.ShapeDtypeStruct((B, D), jnp.float32),
        scratch_types=scratch,
    )(_sc_body)
    return f(table, idx, pos)


def kernel(x, table):
    out = _sc_embed(table, x.astype(jnp.int32), _POS)
    return out.reshape(BATCH, POS_N, D)


# speedup vs baseline: 1.6553x; 1.6553x over previous
import functools

import jax
import jax.numpy as jnp
import numpy as np
from jax import lax
from jax.experimental import pallas as pl
from jax.experimental.pallas import tpu as pltpu
from jax.experimental.pallas import tpu_sc as plsc

VOCAB_N = 100000
D = 1024
POS_N = 2048
BATCH = 4
B = BATCH * POS_N

_info = plsc.get_sparse_core_info()
NC, NS, LANES = _info.num_cores, _info.num_subcores, _info.num_lanes
NW = NC * NS
BPW = B // NW
CH = 16
NCH = BPW // CH
RING = 3
PREF = 2


def _pos_table():
    half = D / 2
    positions = np.arange(POS_N, dtype=np.float32)[:, None]
    depths = np.arange(half, dtype=np.float32)[None, :] / np.float32(half)
    angle_rads = positions * (np.float32(1.0) / (10000.0 ** depths)).astype(np.float32)
    return np.concatenate([np.sin(angle_rads), np.cos(angle_rads)],
                          axis=-1).astype(np.float32)


_POS = _pos_table()


def _sc_body(table_hbm, idx_hbm, pos_hbm, out_hbm, *scr):
    idx_v = scr[0]
    bufs = scr[1:1 + RING]
    pbufs = scr[1 + RING:1 + 2 * RING]
    gsem = scr[1 + 2 * RING:1 + 3 * RING]
    psem = scr[1 + 3 * RING:1 + 4 * RING]
    ssem = scr[1 + 4 * RING:1 + 5 * RING]

    wid = lax.axis_index("s") * NC + lax.axis_index("c")
    base = wid * BPW
    batch = wid // (POS_N // BPW)
    pos_base = (wid % (POS_N // BPW)) * BPW
    pltpu.sync_copy(idx_hbm.at[batch, pl.ds(pos_base, BPW)], idx_v)

    def start_fetch(c):
        s = c % RING
        pltpu.async_copy(
            table_hbm.at[idx_v.at[pl.ds(c * CH, CH)]], bufs[s], gsem[s])
        pltpu.async_copy(pos_hbm.at[pl.ds(pos_base + c * CH, CH)],
                         pbufs[s], psem[s])

    for c in range(PREF):
        start_fetch(c)

    for c in range(NCH):
        s = c % RING
        pltpu.make_async_copy(table_hbm.at[idx_v.at[pl.ds(c * CH, CH)]],
                              bufs[s], gsem[s]).wait()
        pltpu.make_async_copy(
            pos_hbm.at[pl.ds(pos_base + c * CH, CH)],
            pbufs[s], psem[s]).wait()

        buf, pbuf = bufs[s], pbufs[s]

        def row(r, carry, buf=buf, pbuf=pbuf):
            for j in range(D // LANES):
                sl = pl.ds(j * LANES, LANES)
                buf[r, sl] = buf[r, sl] * 32.0 + pbuf[r, sl]
            return carry

        lax.fori_loop(0, CH, row, 0)

        pltpu.async_copy(bufs[s], out_hbm.at[pl.ds(base + c * CH, CH)],
                         ssem[s])
        cp = c + PREF
        if cp < NCH:
            if cp >= RING:
                sp = cp % RING
                pltpu.make_async_copy(
                    bufs[sp], out_hbm.at[pl.ds(base + (cp - RING) * CH, CH)],
                    ssem[sp]).wait()
            start_fetch(cp)

    for c in range(NCH - RING, NCH):
        s = c % RING
        pltpu.make_async_copy(bufs[s], out_hbm.at[pl.ds(base + c * CH, CH)],
                              ssem[s]).wait()


@jax.jit
def _sc_embed(table, idx, pos):
    mesh = plsc.VectorSubcoreMesh(core_axis_name="c", subcore_axis_name="s")
    scratch = ([pltpu.VMEM((BPW,), jnp.int32)]
               + [pltpu.VMEM((CH, D), jnp.float32) for _ in range(RING)]
               + [pltpu.VMEM((CH, D), jnp.float32) for _ in range(RING)]
               + [pltpu.SemaphoreType.DMA for _ in range(3 * RING)])
    f = functools.partial(
        pl.kernel,
        mesh=mesh,
        out_type=jax.ShapeDtypeStruct((B, D), jnp.float32),
        scratch_types=scratch,
    )(_sc_body)
    return f(table, idx, pos)


def kernel(x, table):
    out = _sc_embed(table, x.astype(jnp.int32), _POS)
    return out.reshape(BATCH, POS_N, D)
